# Initial kernel scaffold; baseline (speedup 1.0000x reference)
import jax
import jax.numpy as jnp
from jax.experimental import pallas as pl


def kernel(input_ids, token_embedding_table):
    raise NotImplementedError("write your pallas kernel here")


# speedup vs baseline: 2.6248x; 2.6248x over previous
import functools

import jax
import jax.numpy as jnp
from jax import lax
from jax.experimental import pallas as pl
from jax.experimental.pallas import tpu as pltpu
from jax.experimental.pallas import tpu_sc as plsc

VOCAB = 64
EMBED_DIM = 64
BATCH = 4096
SEQ = 50

_B = BATCH * SEQ
_NW = 32
_B_PER_W = _B // _NW
_CHUNK = 800
_N_CHUNKS = _B_PER_W // _CHUNK


def _sc_gather(table, ids_flat):
    mesh = plsc.VectorSubcoreMesh(core_axis_name="c", subcore_axis_name="s")

    @functools.partial(
        pl.kernel,
        out_type=jax.ShapeDtypeStruct((_B, EMBED_DIM), jnp.float32),
        mesh=mesh,
        scratch_types=[
            pltpu.VMEM((_CHUNK,), jnp.int32),
            pltpu.VMEM((_CHUNK, EMBED_DIM), jnp.float32),
            pltpu.SemaphoreType.DMA,
        ],
        compiler_params=pltpu.CompilerParams(use_tc_tiling_on_sc=False),
    )
    def k(table_hbm, idx_hbm, out_hbm, idx_v, rows_v, sem):
        wid = lax.axis_index("s") * 2 + lax.axis_index("c")
        base = wid * _B_PER_W
        for ch in range(_N_CHUNKS):
            off = base + ch * _CHUNK
            pltpu.sync_copy(idx_hbm.at[pl.ds(off, _CHUNK)], idx_v)
            pltpu.async_copy(table_hbm.at[idx_v], rows_v, sem).wait()
            pltpu.sync_copy(rows_v, out_hbm.at[pl.ds(off, _CHUNK)])

    return k(table, ids_flat)


def kernel(input_ids, token_embedding_table):
    ids_flat = input_ids.reshape(_B)
    out = _sc_gather(token_embedding_table, ids_flat)
    return out.reshape(BATCH, SEQ, EMBED_DIM)


# speedup vs baseline: 2.6301x; 1.0020x over previous
import functools

import jax
import jax.numpy as jnp
from jax import lax
from jax.experimental import pallas as pl
from jax.experimental.pallas import tpu as pltpu
from jax.experimental.pallas import tpu_sc as plsc

VOCAB = 64
EMBED_DIM = 64
BATCH = 4096
SEQ = 50

_B = BATCH * SEQ
_NW = 32
_B_PER_W = _B // _NW
_CHUNK = 800
_N_CHUNKS = _B_PER_W // _CHUNK


def _sc_gather(table, ids_flat):
    mesh = plsc.VectorSubcoreMesh(core_axis_name="c", subcore_axis_name="s")

    @functools.partial(
        pl.kernel,
        out_type=jax.ShapeDtypeStruct((_B, EMBED_DIM), jnp.float32),
        mesh=mesh,
        scratch_types=[
            pltpu.VMEM((_B_PER_W,), jnp.int32),
            pltpu.VMEM((_CHUNK, EMBED_DIM), jnp.float32),
            pltpu.VMEM((_CHUNK, EMBED_DIM), jnp.float32),
            pltpu.SemaphoreType.DMA,
            pltpu.SemaphoreType.DMA,
            pltpu.SemaphoreType.DMA,
            pltpu.SemaphoreType.DMA,
        ],
        compiler_params=pltpu.CompilerParams(use_tc_tiling_on_sc=False),
    )
    def k(table_hbm, idx_hbm, out_hbm, idx_v, rows0, rows1, g0, g1, s0, s1):
        wid = lax.axis_index("s") * 2 + lax.axis_index("c")
        base = wid * _B_PER_W
        rows = [rows0, rows1]
        gsem = [g0, g1]
        ssem = [s0, s1]
        pltpu.sync_copy(idx_hbm.at[pl.ds(base, _B_PER_W)], idx_v)

        def start_gather(ch, b):
            return pltpu.async_copy(
                table_hbm.at[idx_v.at[pl.ds(ch * _CHUNK, _CHUNK)]],
                rows[b], gsem[b])

        gather_d = [None, None]
        store_d = [None, None]
        gather_d[0] = start_gather(0, 0)
        for ch in range(_N_CHUNKS):
            b = ch % 2
            nb = (ch + 1) % 2
            gather_d[b].wait()
            if ch + 1 < _N_CHUNKS:
                if store_d[nb] is not None:
                    store_d[nb].wait()
                gather_d[nb] = start_gather(ch + 1, nb)
            store_d[b] = pltpu.async_copy(
                rows[b], out_hbm.at[pl.ds(base + ch * _CHUNK, _CHUNK)], ssem[b])
        for d in store_d:
            if d is not None:
                d.wait()

    return k(table, ids_flat)


def kernel(input_ids, token_embedding_table):
    ids_flat = input_ids.reshape(_B)
    out = _sc_gather(token_embedding_table, ids_flat)
    return out.reshape(BATCH, SEQ, EMBED_DIM)
